# Initial kernel scaffold; baseline (speedup 1.0000x reference)
import jax
import jax.numpy as jnp
from jax.experimental import pallas as pl


def kernel(x, edge_index, W1, a_src1, a_dst1, b1, W2, a_src2, a_dst2, b2):
    raise NotImplementedError("write your pallas kernel here")


# speedup vs baseline: 22.5072x; 22.5072x over previous
import dataclasses
import functools

import jax
import jax.numpy as jnp
from jax import lax
from jax.experimental import pallas as pl
from jax.experimental.pallas import tpu as pltpu
from jax.experimental.pallas import tpu_sc as plsc

N = 10000
E = 320000
D = 128
DW = 144
NEG_SLOPE = 0.2
EPS = 1e-16
NC = 2
NS = 16
NW = NC * NS
EPW = E // NW
K = 80
NCHUNK = EPW // K
NP = 10240
RPS = NP // NS
BN = 1024
GN = (N + BN - 1) // BN

f32 = jnp.float32
i32 = jnp.int32


def _sc_edge_pass(h, src, dst, a_s, a_d):
    mesh = plsc.VectorSubcoreMesh(core_axis_name="c", subcore_axis_name="s")
    cp = pltpu.CompilerParams()
    if "needs_layout_passes" in pltpu.CompilerParams.__dataclass_fields__:
        cp = dataclasses.replace(cp, needs_layout_passes=False)

    @functools.partial(
        pl.kernel,
        mesh=mesh,
        compiler_params=cp,
        out_type=(jax.ShapeDtypeStruct((NC, NP, D), f32),
                  jax.ShapeDtypeStruct((NC, NS, N), f32)),
        scratch_types=[
            pltpu.VMEM((N,), f32),
            pltpu.VMEM((N,), f32),
            pltpu.VMEM((N,), f32),
            pltpu.VMEM((K,), i32),
            pltpu.VMEM((K,), i32),
            pltpu.VMEM((K, D), f32),
            pltpu.VMEM((32,), f32),
            pltpu.VMEM_SHARED((NP, D), f32),
            pltpu.SemaphoreType.DMA,
        ],
    )
    def edge_kernel(h_hbm, src_hbm, dst_hbm, as_hbm, ad_hbm, acc_hbm, den_hbm,
                    as_tab, ad_tab, den_tab, sidx, didx, rows_g, wtmp,
                    acc_sh, sem):
        c = lax.axis_index("c")
        s = lax.axis_index("s")
        wid = c * NS + s

        pltpu.sync_copy(as_hbm, as_tab)
        pltpu.sync_copy(ad_hbm, ad_tab)

        zero16 = jnp.zeros((16,), f32)

        @pl.loop(0, N, step=16)
        def _(r):
            den_tab[pl.ds(r, 16)] = zero16

        @pl.loop(0, K)
        def _(r):
            @pl.loop(0, D, step=16)
            def _(cc):
                rows_g[r, pl.ds(cc, 16)] = zero16

        @pl.loop(0, RPS, step=K)
        def _(r0):
            pltpu.sync_copy(rows_g, acc_sh.at[pl.ds(s * RPS + r0, K)])

        plsc.subcore_barrier()

        base = wid * EPW
        lane_iota = lax.iota(i32, 16)
        lane_consts = [jnp.full((16,), 16 + t, i32) for t in range(16)]
        lane_masks = [lane_iota == t for t in range(16)]

        @pl.loop(0, NCHUNK)
        def _(i):
            off = base + i * K
            pltpu.sync_copy(src_hbm.at[pl.ds(off, K)], sidx)
            pltpu.sync_copy(dst_hbm.at[pl.ds(off, K)], didx)
            pltpu.async_copy(h_hbm.at[sidx], rows_g, sem).wait()

            @pl.loop(0, K, step=16)
            def _(j):
                sv = sidx[pl.ds(j, 16)]
                dv = didx[pl.ds(j, 16)]
                e = plsc.load_gather(as_tab, [sv]) + plsc.load_gather(ad_tab, [dv])
                e = jnp.where(e > zero16, e, NEG_SLOPE * e)
                w = jnp.exp(e)
                wtmp[pl.ds(16, 16)] = w
                for t in range(16):
                    plsc.addupdate_scatter(den_tab, [dv], w,
                                           mask=lane_masks[t])
                    wv = plsc.load_gather(wtmp, [lane_consts[t]])

                    @pl.loop(0, D, step=16)
                    def _(cc):
                        rows_g[j + t, pl.ds(cc, 16)] = (
                            rows_g[j + t, pl.ds(cc, 16)] * wv)

            pltpu.sync_copy(rows_g, acc_sh.at[didx], add=True)

        plsc.subcore_barrier()

        pltpu.sync_copy(acc_sh.at[pl.ds(s * RPS, RPS)],
                        acc_hbm.at[c, pl.ds(s * RPS, RPS)])
        pltpu.sync_copy(den_tab, den_hbm.at[c, s])

    acc, den = edge_kernel(h, src, dst, a_s, a_d)
    return acc[:, :N, :], den.reshape(NW, N)


def _tc_head(x, W, a_s, a_d):
    def body(x_ref, w_ref, va_ref, vb_ref, h_ref, as_ref, ad_ref):
        h = jnp.dot(x_ref[...], w_ref[...], preferred_element_type=f32)
        h_ref[...] = h
        as_ref[...] = jnp.dot(h, va_ref[...], preferred_element_type=f32)
        ad_ref[...] = jnp.dot(h, vb_ref[...], preferred_element_type=f32)

    return pl.pallas_call(
        body,
        grid=(GN,),
        in_specs=[
            pl.BlockSpec((BN, D), lambda i: (i, 0)),
            pl.BlockSpec((D, D), lambda i: (0, 0)),
            pl.BlockSpec((D,), lambda i: (0,)),
            pl.BlockSpec((D,), lambda i: (0,)),
        ],
        out_specs=[
            pl.BlockSpec((BN, D), lambda i: (i, 0)),
            pl.BlockSpec((BN,), lambda i: (i,)),
            pl.BlockSpec((BN,), lambda i: (i,)),
        ],
        out_shape=[
            jax.ShapeDtypeStruct((N, D), f32),
            jax.ShapeDtypeStruct((N,), f32),
            jax.ShapeDtypeStruct((N,), f32),
        ],
    )(x, W, a_s, a_d)


def _tc_mid(acc0, acc1, den, b, W, a_s, a_d):
    def body(a0_ref, a1_ref, dn_ref, b_ref, w_ref, va_ref, vb_ref,
             h_ref, as_ref, ad_ref):
        acc = a0_ref[...] + a1_ref[...]
        dn = jnp.sum(dn_ref[...], axis=0) + EPS
        xn = acc / dn[:, None] + b_ref[...][None, :]
        h = jnp.dot(xn, w_ref[...], preferred_element_type=f32)
        h_ref[...] = h
        as_ref[...] = jnp.dot(h, va_ref[...], preferred_element_type=f32)
        ad_ref[...] = jnp.dot(h, vb_ref[...], preferred_element_type=f32)

    return pl.pallas_call(
        body,
        grid=(GN,),
        in_specs=[
            pl.BlockSpec((BN, D), lambda i: (i, 0)),
            pl.BlockSpec((BN, D), lambda i: (i, 0)),
            pl.BlockSpec((NW, BN), lambda i: (0, i)),
            pl.BlockSpec((D,), lambda i: (0,)),
            pl.BlockSpec((D, D), lambda i: (0, 0)),
            pl.BlockSpec((D,), lambda i: (0,)),
            pl.BlockSpec((D,), lambda i: (0,)),
        ],
        out_specs=[
            pl.BlockSpec((BN, D), lambda i: (i, 0)),
            pl.BlockSpec((BN,), lambda i: (i,)),
            pl.BlockSpec((BN,), lambda i: (i,)),
        ],
        out_shape=[
            jax.ShapeDtypeStruct((N, D), f32),
            jax.ShapeDtypeStruct((N,), f32),
            jax.ShapeDtypeStruct((N,), f32),
        ],
    )(acc0, acc1, den, b, W, a_s, a_d)


def _tc_final(acc0, acc1, den, b):
    def body(a0_ref, a1_ref, dn_ref, b_ref, o_ref):
        acc = a0_ref[...] + a1_ref[...]
        dn = jnp.sum(dn_ref[...], axis=0) + EPS
        o_ref[...] = acc / dn[:, None] + b_ref[...][None, :]

    return pl.pallas_call(
        body,
        grid=(GN,),
        in_specs=[
            pl.BlockSpec((BN, D), lambda i: (i, 0)),
            pl.BlockSpec((BN, D), lambda i: (i, 0)),
            pl.BlockSpec((NW, BN), lambda i: (0, i)),
            pl.BlockSpec((D,), lambda i: (0,)),
        ],
        out_specs=pl.BlockSpec((BN, D), lambda i: (i, 0)),
        out_shape=jax.ShapeDtypeStruct((N, D), f32),
    )(acc0, acc1, den, b)


def kernel(x, edge_index, W1, a_src1, a_dst1, b1, W2, a_src2, a_dst2, b2):
    src = edge_index[0]
    dst = edge_index[1]
    h1, as1, ad1 = _tc_head(x, W1, a_src1, a_dst1)
    acc1, den1 = _sc_edge_pass(h1, src, dst, as1, ad1)
    h2, as2, ad2 = _tc_mid(acc1[0], acc1[1], den1, b1, W2, a_src2, a_dst2)
    acc2, den2 = _sc_edge_pass(h2, src, dst, as2, ad2)
    return _tc_final(acc2[0], acc2[1], den2, b2)


# speedup vs baseline: 31.5758x; 1.4029x over previous
import dataclasses
import functools

import jax
import jax.numpy as jnp
from jax import lax
from jax.experimental import pallas as pl
from jax.experimental.pallas import tpu as pltpu
from jax.experimental.pallas import tpu_sc as plsc

N = 10000
E = 320000
D = 128
DW = 144
NEG_SLOPE = 0.2
EPS = 1e-16
NC = 2
NS = 16
NW = NC * NS
EPW = E // NW
K = 80
NCHUNK = EPW // K
NP = 10240
RPS = NP // NS
BN = 1024
GN = (N + BN - 1) // BN

f32 = jnp.float32
i32 = jnp.int32


def _sc_edge_pass(h, src, dst, a_s, a_d):
    mesh = plsc.VectorSubcoreMesh(core_axis_name="c", subcore_axis_name="s")
    cp = pltpu.CompilerParams()
    if "needs_layout_passes" in pltpu.CompilerParams.__dataclass_fields__:
        cp = dataclasses.replace(cp, needs_layout_passes=False)

    @functools.partial(
        pl.kernel,
        mesh=mesh,
        compiler_params=cp,
        out_type=(jax.ShapeDtypeStruct((NC, NP, D), f32),
                  jax.ShapeDtypeStruct((NC, NS, N), f32)),
        scratch_types=[
            pltpu.VMEM((N,), f32),
            pltpu.VMEM((2, K), i32),
            pltpu.VMEM((2, K), i32),
            pltpu.VMEM((2, K), f32),
            pltpu.VMEM((2, K), f32),
            pltpu.VMEM((2, K, D), f32),
            pltpu.VMEM((32,), f32),
            pltpu.VMEM_SHARED((NP, D), f32),
            pltpu.SemaphoreType.DMA,
            pltpu.SemaphoreType.DMA,
            pltpu.SemaphoreType.DMA,
            pltpu.SemaphoreType.DMA,
        ],
    )
    def edge_kernel(h_hbm, src_hbm, dst_hbm, as_hbm, ad_hbm, acc_hbm, den_hbm,
                    den_tab, sidx2, didx2, asv2, adv2, rows2, wtmp,
                    acc_sh, semg0, semg1, semc0, semc1):
        c = lax.axis_index("c")
        s = lax.axis_index("s")
        wid = c * NS + s
        base = wid * EPW
        semg = [semg0, semg1]
        semc = [semc0, semc1]

        zero16 = jnp.zeros((16,), f32)
        lane_iota = lax.iota(i32, 16)
        lane_consts = [jnp.full((16,), 16 + t, i32) for t in range(16)]
        lane_masks = [lane_iota == t for t in range(16)]

        @pl.loop(0, N, step=16)
        def _(r):
            den_tab[pl.ds(r, 16)] = zero16

        @pl.loop(0, K)
        def _(r):
            @pl.loop(0, D, step=16)
            def _(cc):
                rows2[0, r, pl.ds(cc, 16)] = zero16

        @pl.loop(0, RPS, step=K)
        def _(r0):
            pltpu.sync_copy(rows2.at[0], acc_sh.at[pl.ds(s * RPS + r0, K)])

        def fetch_idx(off, bb):
            pltpu.sync_copy(src_hbm.at[pl.ds(off, K)], sidx2.at[bb])
            pltpu.sync_copy(dst_hbm.at[pl.ds(off, K)], didx2.at[bb])

        def gather_start(bb):
            pltpu.async_copy(h_hbm.at[sidx2.at[bb]], rows2.at[bb], semg[bb])
            pltpu.async_copy(as_hbm.at[sidx2.at[bb]], asv2.at[bb], semg[bb])
            pltpu.async_copy(ad_hbm.at[didx2.at[bb]], adv2.at[bb], semg[bb])

        def gather_wait(bb):
            pltpu.make_async_copy(h_hbm.at[sidx2.at[bb]], rows2.at[bb],
                                  semg[bb]).wait()
            pltpu.make_async_copy(as_hbm.at[sidx2.at[bb]], asv2.at[bb],
                                  semg[bb]).wait()
            pltpu.make_async_copy(ad_hbm.at[didx2.at[bb]], adv2.at[bb],
                                  semg[bb]).wait()

        def scatter_start(bb):
            pltpu.async_copy(rows2.at[bb], acc_sh.at[didx2.at[bb]], semc[bb],
                             add=True)

        def scatter_wait(bb):
            pltpu.make_async_copy(rows2.at[bb], acc_sh.at[didx2.at[bb]],
                                  semc[bb]).wait()

        def chunk_step(ib, b, wait_prev, do_pf):
            nb = 1 - b
            gather_wait(b)
            if wait_prev:
                scatter_wait(nb)
            if do_pf:
                fetch_idx(base + (ib + 1) * K, nb)
                gather_start(nb)

            @pl.loop(0, K, step=16)
            def _(j):
                dv = didx2[b, pl.ds(j, 16)]
                e = asv2[b, pl.ds(j, 16)] + adv2[b, pl.ds(j, 16)]
                e = jnp.where(e > zero16, e, NEG_SLOPE * e)
                w = jnp.exp(e)
                wtmp[pl.ds(16, 16)] = w
                for t in range(16):
                    plsc.addupdate_scatter(den_tab, [dv], w,
                                           mask=lane_masks[t])
                    wv = plsc.load_gather(wtmp, [lane_consts[t]])
                    for cc in range(0, D, 16):
                        rows2[b, j + t, pl.ds(cc, 16)] = (
                            rows2[b, j + t, pl.ds(cc, 16)] * wv)

            scatter_start(b)

        fetch_idx(base, 0)
        gather_start(0)
        plsc.subcore_barrier()

        chunk_step(0, 0, False, True)

        @pl.loop(1, NCHUNK - 2, step=2)
        def _(ib):
            chunk_step(ib, 1, True, True)
            chunk_step(ib + 1, 0, True, True)

        chunk_step(NCHUNK - 2, 1, True, True)
        chunk_step(NCHUNK - 1, 0, True, False)
        scatter_wait(0)

        plsc.subcore_barrier()

        pltpu.sync_copy(acc_sh.at[pl.ds(s * RPS, RPS)],
                        acc_hbm.at[c, pl.ds(s * RPS, RPS)])
        pltpu.sync_copy(den_tab, den_hbm.at[c, s])

    acc, den = edge_kernel(h, src, dst, a_s, a_d)
    return acc[:, :N, :], den.reshape(NW, N)


def _tc_head(x, W, a_s, a_d):
    def body(x_ref, w_ref, va_ref, vb_ref, h_ref, as_ref, ad_ref):
        h = jnp.dot(x_ref[...], w_ref[...], preferred_element_type=f32)
        h_ref[...] = h
        as_ref[...] = jnp.dot(h, va_ref[...], preferred_element_type=f32)
        ad_ref[...] = jnp.dot(h, vb_ref[...], preferred_element_type=f32)

    return pl.pallas_call(
        body,
        grid=(GN,),
        in_specs=[
            pl.BlockSpec((BN, D), lambda i: (i, 0)),
            pl.BlockSpec((D, D), lambda i: (0, 0)),
            pl.BlockSpec((D,), lambda i: (0,)),
            pl.BlockSpec((D,), lambda i: (0,)),
        ],
        out_specs=[
            pl.BlockSpec((BN, D), lambda i: (i, 0)),
            pl.BlockSpec((BN,), lambda i: (i,)),
            pl.BlockSpec((BN,), lambda i: (i,)),
        ],
        out_shape=[
            jax.ShapeDtypeStruct((N, D), f32),
            jax.ShapeDtypeStruct((N,), f32),
            jax.ShapeDtypeStruct((N,), f32),
        ],
    )(x, W, a_s, a_d)


def _tc_mid(acc0, acc1, den, b, W, a_s, a_d):
    def body(a0_ref, a1_ref, dn_ref, b_ref, w_ref, va_ref, vb_ref,
             h_ref, as_ref, ad_ref):
        acc = a0_ref[...] + a1_ref[...]
        dn = jnp.sum(dn_ref[...], axis=0) + EPS
        xn = acc / dn[:, None] + b_ref[...][None, :]
        h = jnp.dot(xn, w_ref[...], preferred_element_type=f32)
        h_ref[...] = h
        as_ref[...] = jnp.dot(h, va_ref[...], preferred_element_type=f32)
        ad_ref[...] = jnp.dot(h, vb_ref[...], preferred_element_type=f32)

    return pl.pallas_call(
        body,
        grid=(GN,),
        in_specs=[
            pl.BlockSpec((BN, D), lambda i: (i, 0)),
            pl.BlockSpec((BN, D), lambda i: (i, 0)),
            pl.BlockSpec((NW, BN), lambda i: (0, i)),
            pl.BlockSpec((D,), lambda i: (0,)),
            pl.BlockSpec((D, D), lambda i: (0, 0)),
            pl.BlockSpec((D,), lambda i: (0,)),
            pl.BlockSpec((D,), lambda i: (0,)),
        ],
        out_specs=[
            pl.BlockSpec((BN, D), lambda i: (i, 0)),
            pl.BlockSpec((BN,), lambda i: (i,)),
            pl.BlockSpec((BN,), lambda i: (i,)),
        ],
        out_shape=[
            jax.ShapeDtypeStruct((N, D), f32),
            jax.ShapeDtypeStruct((N,), f32),
            jax.ShapeDtypeStruct((N,), f32),
        ],
    )(acc0, acc1, den, b, W, a_s, a_d)


def _tc_final(acc0, acc1, den, b):
    def body(a0_ref, a1_ref, dn_ref, b_ref, o_ref):
        acc = a0_ref[...] + a1_ref[...]
        dn = jnp.sum(dn_ref[...], axis=0) + EPS
        o_ref[...] = acc / dn[:, None] + b_ref[...][None, :]

    return pl.pallas_call(
        body,
        grid=(GN,),
        in_specs=[
            pl.BlockSpec((BN, D), lambda i: (i, 0)),
            pl.BlockSpec((BN, D), lambda i: (i, 0)),
            pl.BlockSpec((NW, BN), lambda i: (0, i)),
            pl.BlockSpec((D,), lambda i: (0,)),
        ],
        out_specs=pl.BlockSpec((BN, D), lambda i: (i, 0)),
        out_shape=jax.ShapeDtypeStruct((N, D), f32),
    )(acc0, acc1, den, b)


def kernel(x, edge_index, W1, a_src1, a_dst1, b1, W2, a_src2, a_dst2, b2):
    src = edge_index[0]
    dst = edge_index[1]
    h1, as1, ad1 = _tc_head(x, W1, a_src1, a_dst1)
    acc1, den1 = _sc_edge_pass(h1, src, dst, as1, ad1)
    h2, as2, ad2 = _tc_mid(acc1[0], acc1[1], den1, b1, W2, a_src2, a_dst2)
    acc2, den2 = _sc_edge_pass(h2, src, dst, as2, ad2)
    return _tc_final(acc2[0], acc2[1], den2, b2)


# speedup vs baseline: 43.6965x; 1.3839x over previous
import dataclasses
import functools

import jax
import jax.numpy as jnp
from jax import lax
from jax.experimental import pallas as pl
from jax.experimental.pallas import tpu as pltpu
from jax.experimental.pallas import tpu_sc as plsc

N = 10000
E = 320000
D = 128
DW = 144
NEG_SLOPE = 0.2
EPS = 1e-16
NC = 2
NS = 16
NW = NC * NS
EPW = E // NW
K = 80
NCHUNK = EPW // K
PH = 25
NPHASE = NCHUNK // PH
NP = 10240
RPS = NP // NS
BN = 1024
GN = (N + BN - 1) // BN

f32 = jnp.float32
i32 = jnp.int32


def _sc_edge_pass(h, src, dst, a_s, a_d):
    mesh = plsc.VectorSubcoreMesh(core_axis_name="c", subcore_axis_name="s")
    cp = pltpu.CompilerParams()
    if "needs_layout_passes" in pltpu.CompilerParams.__dataclass_fields__:
        cp = dataclasses.replace(cp, needs_layout_passes=False)

    @functools.partial(
        pl.kernel,
        mesh=mesh,
        compiler_params=cp,
        out_type=(jax.ShapeDtypeStruct((NC, NP, D), f32),
                  jax.ShapeDtypeStruct((NC, NS, N), f32)),
        scratch_types=[
            pltpu.VMEM((N,), f32),
            pltpu.VMEM((PH, K), i32),
            pltpu.VMEM((PH, K), i32),
            pltpu.VMEM((2, K), f32),
            pltpu.VMEM((2, K), f32),
            pltpu.VMEM((2, K, D), f32),
            pltpu.VMEM((32,), f32),
            pltpu.VMEM_SHARED((NP, D), f32),
            pltpu.SemaphoreType.DMA,
            pltpu.SemaphoreType.DMA,
            pltpu.SemaphoreType.DMA,
            pltpu.SemaphoreType.DMA,
        ],
    )
    def edge_kernel(h_hbm, src_hbm, dst_hbm, as_hbm, ad_hbm, acc_hbm, den_hbm,
                    den_tab, sph, dph, asv2, adv2, rows2, wtmp,
                    acc_sh, semg0, semg1, semc0, semc1):
        c = lax.axis_index("c")
        s = lax.axis_index("s")
        wid = c * NS + s
        semg = [semg0, semg1]
        semc = [semc0, semc1]

        zero16 = jnp.zeros((16,), f32)
        lane_iota = lax.iota(i32, 16)
        lane_consts = [jnp.full((16,), 16 + t, i32) for t in range(16)]
        lane_masks = [lane_iota == t for t in range(16)]

        @pl.loop(0, N, step=16)
        def _(r):
            den_tab[pl.ds(r, 16)] = zero16

        @pl.loop(0, K)
        def _(r):
            @pl.loop(0, D, step=16)
            def _(cc):
                rows2[0, r, pl.ds(cc, 16)] = zero16

        @pl.loop(0, RPS, step=K)
        def _(r0):
            pltpu.sync_copy(rows2.at[0], acc_sh.at[pl.ds(s * RPS + r0, K)])

        def gather_start(r, bb):
            pltpu.async_copy(h_hbm.at[sph.at[r]], rows2.at[bb], semg[bb])
            pltpu.async_copy(as_hbm.at[sph.at[r]], asv2.at[bb], semg[bb])
            pltpu.async_copy(ad_hbm.at[dph.at[r]], adv2.at[bb], semg[bb])

        def gather_wait(r, bb):
            pltpu.make_async_copy(h_hbm.at[sph.at[r]], rows2.at[bb],
                                  semg[bb]).wait()
            pltpu.make_async_copy(as_hbm.at[sph.at[r]], asv2.at[bb],
                                  semg[bb]).wait()
            pltpu.make_async_copy(ad_hbm.at[dph.at[r]], adv2.at[bb],
                                  semg[bb]).wait()

        def scatter_start(r, bb):
            pltpu.async_copy(rows2.at[bb], acc_sh.at[dph.at[r]], semc[bb],
                             add=True)

        def scatter_wait(r, bb):
            pltpu.make_async_copy(rows2.at[bb], acc_sh.at[dph.at[r]],
                                  semc[bb]).wait()

        def chunk_step(r, b, prev_r, do_pf):
            nb = 1 - b
            gather_wait(r, b)
            if prev_r is not None:
                scatter_wait(prev_r, nb)
            if do_pf:
                gather_start(r + 1, nb)

            @pl.loop(0, K, step=16)
            def _(j):
                dv = dph[r, pl.ds(j, 16)]
                e = asv2[b, pl.ds(j, 16)] + adv2[b, pl.ds(j, 16)]
                e = jnp.where(e > zero16, e, NEG_SLOPE * e)
                w = jnp.exp(e)
                wtmp[pl.ds(16, 16)] = w
                for t in range(16):
                    plsc.addupdate_scatter(den_tab, [dv], w,
                                           mask=lane_masks[t])
                    wv = plsc.load_gather(wtmp, [lane_consts[t]])
                    for cc in range(0, D, 16):
                        rows2[b, j + t, pl.ds(cc, 16)] = (
                            rows2[b, j + t, pl.ds(cc, 16)] * wv)

            scatter_start(r, b)

        plsc.subcore_barrier()

        @pl.loop(0, NPHASE)
        def _(p):
            pltpu.sync_copy(src_hbm.at[wid, p], sph)
            pltpu.sync_copy(dst_hbm.at[wid, p], dph)
            gather_start(0, 0)
            chunk_step(0, 0, None, True)

            @pl.loop(1, PH - 2, step=2)
            def _(r):
                chunk_step(r, 1, r - 1, True)
                chunk_step(r + 1, 0, r, True)

            chunk_step(PH - 2, 1, PH - 3, True)
            chunk_step(PH - 1, 0, PH - 2, False)
            scatter_wait(PH - 1, 0)

        plsc.subcore_barrier()

        pltpu.sync_copy(acc_sh.at[pl.ds(s * RPS, RPS)],
                        acc_hbm.at[c, pl.ds(s * RPS, RPS)])
        pltpu.sync_copy(den_tab, den_hbm.at[c, s])

    src4 = src.reshape(NW, NPHASE, PH, K)
    dst4 = dst.reshape(NW, NPHASE, PH, K)
    acc, den = edge_kernel(h, src4, dst4, a_s, a_d)
    return acc[:, :N, :], den.reshape(NW, N)


def _tc_head(x, W, a_s, a_d):
    def body(x_ref, w_ref, va_ref, vb_ref, h_ref, as_ref, ad_ref):
        h = jnp.dot(x_ref[...], w_ref[...], preferred_element_type=f32)
        h_ref[...] = h
        as_ref[...] = jnp.dot(h, va_ref[...], preferred_element_type=f32)
        ad_ref[...] = jnp.dot(h, vb_ref[...], preferred_element_type=f32)

    return pl.pallas_call(
        body,
        grid=(GN,),
        in_specs=[
            pl.BlockSpec((BN, D), lambda i: (i, 0)),
            pl.BlockSpec((D, D), lambda i: (0, 0)),
            pl.BlockSpec((D,), lambda i: (0,)),
            pl.BlockSpec((D,), lambda i: (0,)),
        ],
        out_specs=[
            pl.BlockSpec((BN, D), lambda i: (i, 0)),
            pl.BlockSpec((BN,), lambda i: (i,)),
            pl.BlockSpec((BN,), lambda i: (i,)),
        ],
        out_shape=[
            jax.ShapeDtypeStruct((N, D), f32),
            jax.ShapeDtypeStruct((N,), f32),
            jax.ShapeDtypeStruct((N,), f32),
        ],
    )(x, W, a_s, a_d)


def _tc_mid(acc0, acc1, den, b, W, a_s, a_d):
    def body(a0_ref, a1_ref, dn_ref, b_ref, w_ref, va_ref, vb_ref,
             h_ref, as_ref, ad_ref):
        acc = a0_ref[...] + a1_ref[...]
        dn = jnp.sum(dn_ref[...], axis=0) + EPS
        xn = acc / dn[:, None] + b_ref[...][None, :]
        h = jnp.dot(xn, w_ref[...], preferred_element_type=f32)
        h_ref[...] = h
        as_ref[...] = jnp.dot(h, va_ref[...], preferred_element_type=f32)
        ad_ref[...] = jnp.dot(h, vb_ref[...], preferred_element_type=f32)

    return pl.pallas_call(
        body,
        grid=(GN,),
        in_specs=[
            pl.BlockSpec((BN, D), lambda i: (i, 0)),
            pl.BlockSpec((BN, D), lambda i: (i, 0)),
            pl.BlockSpec((NW, BN), lambda i: (0, i)),
            pl.BlockSpec((D,), lambda i: (0,)),
            pl.BlockSpec((D, D), lambda i: (0, 0)),
            pl.BlockSpec((D,), lambda i: (0,)),
            pl.BlockSpec((D,), lambda i: (0,)),
        ],
        out_specs=[
            pl.BlockSpec((BN, D), lambda i: (i, 0)),
            pl.BlockSpec((BN,), lambda i: (i,)),
            pl.BlockSpec((BN,), lambda i: (i,)),
        ],
        out_shape=[
            jax.ShapeDtypeStruct((N, D), f32),
            jax.ShapeDtypeStruct((N,), f32),
            jax.ShapeDtypeStruct((N,), f32),
        ],
    )(acc0, acc1, den, b, W, a_s, a_d)


def _tc_final(acc0, acc1, den, b):
    def body(a0_ref, a1_ref, dn_ref, b_ref, o_ref):
        acc = a0_ref[...] + a1_ref[...]
        dn = jnp.sum(dn_ref[...], axis=0) + EPS
        o_ref[...] = acc / dn[:, None] + b_ref[...][None, :]

    return pl.pallas_call(
        body,
        grid=(GN,),
        in_specs=[
            pl.BlockSpec((BN, D), lambda i: (i, 0)),
            pl.BlockSpec((BN, D), lambda i: (i, 0)),
            pl.BlockSpec((NW, BN), lambda i: (0, i)),
            pl.BlockSpec((D,), lambda i: (0,)),
        ],
        out_specs=pl.BlockSpec((BN, D), lambda i: (i, 0)),
        out_shape=jax.ShapeDtypeStruct((N, D), f32),
    )(acc0, acc1, den, b)


def kernel(x, edge_index, W1, a_src1, a_dst1, b1, W2, a_src2, a_dst2, b2):
    src = edge_index[0]
    dst = edge_index[1]
    h1, as1, ad1 = _tc_head(x, W1, a_src1, a_dst1)
    acc1, den1 = _sc_edge_pass(h1, src, dst, as1, ad1)
    h2, as2, ad2 = _tc_mid(acc1[0], acc1[1], den1, b1, W2, a_src2, a_dst2)
    acc2, den2 = _sc_edge_pass(h2, src, dst, as2, ad2)
    return _tc_final(acc2[0], acc2[1], den2, b2)
